# baseline (speedup 1.0000x reference)
import functools

import jax
import jax.numpy as jnp
from jax import lax
from jax.experimental import pallas as pl
from jax.experimental.pallas import tpu as pltpu
from jax.experimental.pallas import tpu_sc as plsc

Q, K, D = 4096, 16384, 3
KNN = 32
HID = 1024
N_OUT, EV = 32, 2

QB = 128
QB2 = 512


def _topk_body(q_ref, kt_ref, q2_ref, k2_ref, od_ref, oi_ref, dist_ref):
    q = q_ref[...]
    kt = kt_ref[...]
    qk = jax.lax.dot_general(q, kt, (((1,), (0,)), ((), ())),
                             preferred_element_type=jnp.float32)
    q2 = q2_ref[...]
    k2 = k2_ref[0:1, :]
    dist_ref[...] = (q2 + k2) - 2.0 * qk

    iota = lax.broadcasted_iota(jnp.int32, (QB, K), 1)
    lane32 = lax.broadcasted_iota(jnp.int32, (QB, KNN), 1)
    inf = jnp.float32(jnp.inf)

    def body(j, carry):
        od, oi, lm, li = carry
        d = dist_ref[...]
        live = (d > lm[:, None]) | ((d == lm[:, None]) & (iota > li[:, None]))
        dc = jnp.where(live, d, inf)
        m = jnp.min(dc, axis=1)
        idx = jnp.min(jnp.where(dc == m[:, None], iota, K), axis=1)
        od = jnp.where(lane32 == j, m[:, None], od)
        oi = jnp.where(lane32 == j, idx[:, None], oi)
        return od, oi, m, idx

    od0 = jnp.zeros((QB, KNN), jnp.float32)
    oi0 = jnp.zeros((QB, KNN), jnp.int32)
    lm0 = jnp.full((QB,), -jnp.inf, jnp.float32)
    li0 = jnp.full((QB,), -1, jnp.int32)
    od, oi, _, _ = lax.fori_loop(0, KNN, body, (od0, oi0, lm0, li0))
    od_ref[...] = od
    oi_ref[...] = oi


def _topk(queries, keys_t, q2, k2b):
    return pl.pallas_call(
        _topk_body,
        grid=(Q // QB,),
        in_specs=[
            pl.BlockSpec((QB, D), lambda i: (i, 0)),
            pl.BlockSpec((D, K), lambda i: (0, 0)),
            pl.BlockSpec((QB, 1), lambda i: (i, 0)),
            pl.BlockSpec((8, K), lambda i: (0, 0)),
        ],
        out_specs=[
            pl.BlockSpec((QB, KNN), lambda i: (i, 0)),
            pl.BlockSpec((QB, KNN), lambda i: (i, 0)),
        ],
        out_shape=[
            jax.ShapeDtypeStruct((Q, KNN), jnp.float32),
            jax.ShapeDtypeStruct((Q, KNN), jnp.int32),
        ],
        scratch_shapes=[pltpu.VMEM((QB, K), jnp.float32)],
    )(queries, keys_t, q2, k2b)


def _sc_gather(keys_flat, k_inds, queries):
    info = plsc.get_sparse_core_info()
    nc, ns = info.num_cores, info.num_subcores
    nw = nc * ns
    qw = Q // nw
    mesh = plsc.VectorSubcoreMesh(core_axis_name="c", subcore_axis_name="s")

    @functools.partial(
        pl.kernel, mesh=mesh,
        compiler_params=pltpu.CompilerParams(needs_layout_passes=False),
        out_type=jax.ShapeDtypeStruct((Q * D * KNN,), jnp.float32),
        scratch_types=[
            pltpu.VMEM((K * D,), jnp.float32),
            pltpu.VMEM((qw, KNN), jnp.int32),
            pltpu.VMEM((qw * D + 16,), jnp.float32),
            pltpu.VMEM((qw * D * KNN,), jnp.float32),
        ],
    )
    def k(keys_hbm, kinds_hbm, q_hbm, out_hbm, keys_v, kinds_v, q_v, out_v):
        wid = lax.axis_index("s") * nc + lax.axis_index("c")
        base = wid * qw
        pltpu.sync_copy(keys_hbm, keys_v)
        pltpu.sync_copy(kinds_hbm.at[pl.ds(base, qw)], kinds_v)
        pltpu.sync_copy(q_hbm.at[pl.ds(base * D, qw * D)],
                        q_v.at[pl.ds(0, qw * D)])

        lane = lax.iota(jnp.int32, 16)

        def body(qi, c):
            qrow = q_v[pl.ds(qi * D, 16)]
            obase = lane + qi * (D * KNN)
            for g in range(KNN // 16):
                kv = kinds_v[qi, pl.ds(16 * g, 16)]
                kv3 = kv * 3
                for d in range(D):
                    val = plsc.load_gather(keys_v, [kv3 + d])
                    plsc.store_scatter(out_v, [obase + (d * KNN + 16 * g)],
                                       val - qrow[d])
            return c

        lax.fori_loop(0, qw, body, 0)
        pltpu.sync_copy(out_v, out_hbm.at[pl.ds(base * D * KNN, qw * D * KNN)])

    return k(keys_flat, k_inds, queries)


def _mlp_body(x_ref, w0_ref, b0_ref, w1_ref, b1_ref, w2_ref, b2_ref,
              wo_ref, bo_ref, o_ref):
    h = jnp.maximum(jnp.dot(x_ref[...], w0_ref[...],
                            preferred_element_type=jnp.float32) + b0_ref[...], 0.0)
    h = jnp.maximum(jnp.dot(h, w1_ref[...],
                            preferred_element_type=jnp.float32) + b1_ref[...], 0.0)
    h = jnp.maximum(jnp.dot(h, w2_ref[...],
                            preferred_element_type=jnp.float32) + b2_ref[...], 0.0)
    o_ref[...] = jnp.dot(h, wo_ref[...],
                         preferred_element_type=jnp.float32) + bo_ref[...]


def _mlp(x, w0, b0, w1, b1, w2, b2, wo, bo):
    full = lambda shape: pl.BlockSpec(shape, lambda i: tuple(0 for _ in shape))
    return pl.pallas_call(
        _mlp_body,
        grid=(Q // QB2,),
        in_specs=[
            pl.BlockSpec((QB2, KNN * D), lambda i: (i, 0)),
            full((KNN * D, HID)), full((HID,)),
            full((HID, HID)), full((HID,)),
            full((HID, HID)), full((HID,)),
            full((HID, N_OUT * D * EV)), full((N_OUT * D * EV,)),
        ],
        out_specs=pl.BlockSpec((QB2, N_OUT * D * EV), lambda i: (i, 0)),
        out_shape=jax.ShapeDtypeStruct((Q, N_OUT * D * EV), jnp.float32),
    )(x, w0, b0, w1, b1, w2, b2, wo, bo)


def kernel(queries, keys, W0, b0, W1, b1, W2, b2, Wout, bout):
    q2 = jnp.sum(queries * queries, axis=-1, keepdims=True)
    k2 = jnp.sum(keys * keys, axis=-1)
    k2b = jnp.broadcast_to(k2[None, :], (8, K))
    k_dists, k_inds = _topk(queries, keys.T, q2, k2b)
    flat_t = _sc_gather(keys.reshape(-1), k_inds,
                        queries.reshape(-1)).reshape(Q, D * KNN)
    W0_t = W0.reshape(KNN, D, HID).transpose(1, 0, 2).reshape(KNN * D, HID)
    out = _mlp(flat_t, W0_t, b0, W1, b1, W2, b2, Wout, bout)
    params = out.reshape(Q, N_OUT, D, EV)
    return params, k_inds, k_dists


# speedup vs baseline: 1.2202x; 1.2202x over previous
import functools

import jax
import jax.numpy as jnp
from jax import lax
from jax.experimental import pallas as pl
from jax.experimental.pallas import tpu as pltpu
from jax.experimental.pallas import tpu_sc as plsc

Q, K, D = 4096, 16384, 3
KNN = 32
HID = 1024
N_OUT, EV = 32, 2

QB = 128
QB2 = 512


CS = 512
NC_ = K // CS
TPC = 8
NCAND = NC_ * TPC


def _dist_into(q_ref, kt_ref, q2_ref, k2_ref, dist_ref):
    q = q_ref[...]
    kt = kt_ref[...]
    qk = jax.lax.dot_general(q, kt, (((1,), (0,)), ((), ())),
                             preferred_element_type=jnp.float32)
    q2 = q2_ref[...]
    k2 = k2_ref[0:1, :]
    dist_ref[...] = (q2 + k2) - 2.0 * qk


def _topk_body(q_ref, kt_ref, q2_ref, k2_ref, od_ref, oi_ref,
               dist_ref, cd_ref, ci_ref):
    _dist_into(q_ref, kt_ref, q2_ref, k2_ref, dist_ref)

    inf = jnp.float32(jnp.inf)
    lane32 = lax.broadcasted_iota(jnp.int32, (QB, KNN), 1)

    iota_s = lax.broadcasted_iota(jnp.int32, (QB, CS), 1)
    lane_t = lax.broadcasted_iota(jnp.int32, (QB, TPC), 1)

    for c in range(NC_):
        def round_body(t, tc, c=c):
            md, mi = tc
            d = dist_ref[:, c * CS:(c + 1) * CS]
            m = jnp.min(d, axis=1)
            li = jnp.min(jnp.where(d == m[:, None], iota_s, CS), axis=1)
            dist_ref[:, c * CS:(c + 1) * CS] = \
                jnp.where(iota_s == li[:, None], inf, d)
            md = jnp.where(lane_t == t, m[:, None], md)
            mi = jnp.where(lane_t == t, (li + c * CS)[:, None], mi)
            return md, mi

        md0 = jnp.zeros((QB, TPC), jnp.float32)
        mi0 = jnp.zeros((QB, TPC), jnp.int32)
        md, mi = lax.fori_loop(0, TPC, round_body, (md0, mi0))
        cd_ref[:, c * TPC:(c + 1) * TPC] = md
        ci_ref[:, c * TPC:(c + 1) * TPC] = mi

    iota_c = lax.broadcasted_iota(jnp.int32, (QB, NCAND), 1)
    chunk_iota = lax.broadcasted_iota(jnp.int32, (QB, NC_), 1)

    def mbody(j, carry):
        od, oi, counts = carry
        cd = cd_ref[...]
        m = jnp.min(cd, axis=1)
        pos = jnp.min(jnp.where(cd == m[:, None], iota_c, NCAND), axis=1)
        gi = jnp.min(jnp.where(iota_c == pos[:, None], ci_ref[...], K), axis=1)
        cd_ref[...] = jnp.where(iota_c == pos[:, None], inf, cd)
        od = jnp.where(lane32 == j, m[:, None], od)
        oi = jnp.where(lane32 == j, gi[:, None], oi)
        counts = counts + (chunk_iota == (pos // TPC)[:, None]).astype(jnp.int32)
        return od, oi, counts

    od0 = jnp.zeros((QB, KNN), jnp.float32)
    oi0 = jnp.zeros((QB, KNN), jnp.int32)
    cnt0 = jnp.zeros((QB, NC_), jnp.int32)
    od, oi, counts = lax.fori_loop(0, KNN, mbody, (od0, oi0, cnt0))
    od_ref[...] = od
    oi_ref[...] = oi

    suspect = jnp.max(jnp.where(counts >= TPC, 1, 0))

    @pl.when(suspect == 1)
    def _fallback():
        _dist_into(q_ref, kt_ref, q2_ref, k2_ref, dist_ref)
        iota = lax.broadcasted_iota(jnp.int32, (QB, K), 1)

        def fb(j, carry):
            fod, foi = carry
            d = dist_ref[...]
            m = jnp.min(d, axis=1)
            idx = jnp.min(jnp.where(d == m[:, None], iota, K), axis=1)
            dist_ref[...] = jnp.where(iota == idx[:, None], inf, d)
            fod = jnp.where(lane32 == j, m[:, None], fod)
            foi = jnp.where(lane32 == j, idx[:, None], foi)
            return fod, foi

        fod, foi = lax.fori_loop(0, KNN, fb, (od0, oi0))
        od_ref[...] = fod
        oi_ref[...] = foi


def _topk(queries, keys_t, q2, k2b):
    return pl.pallas_call(
        _topk_body,
        grid=(Q // QB,),
        in_specs=[
            pl.BlockSpec((QB, D), lambda i: (i, 0)),
            pl.BlockSpec((D, K), lambda i: (0, 0)),
            pl.BlockSpec((QB, 1), lambda i: (i, 0)),
            pl.BlockSpec((8, K), lambda i: (0, 0)),
        ],
        out_specs=[
            pl.BlockSpec((QB, KNN), lambda i: (i, 0)),
            pl.BlockSpec((QB, KNN), lambda i: (i, 0)),
        ],
        out_shape=[
            jax.ShapeDtypeStruct((Q, KNN), jnp.float32),
            jax.ShapeDtypeStruct((Q, KNN), jnp.int32),
        ],
        scratch_shapes=[
            pltpu.VMEM((QB, K), jnp.float32),
            pltpu.VMEM((QB, NCAND), jnp.float32),
            pltpu.VMEM((QB, NCAND), jnp.int32),
        ],
    )(queries, keys_t, q2, k2b)


def _sc_gather(keys_flat, k_inds, queries):
    info = plsc.get_sparse_core_info()
    nc, ns = info.num_cores, info.num_subcores
    nw = nc * ns
    qw = Q // nw
    mesh = plsc.VectorSubcoreMesh(core_axis_name="c", subcore_axis_name="s")

    @functools.partial(
        pl.kernel, mesh=mesh,
        compiler_params=pltpu.CompilerParams(needs_layout_passes=False),
        out_type=jax.ShapeDtypeStruct((Q * D * KNN,), jnp.float32),
        scratch_types=[
            pltpu.VMEM((K * D,), jnp.float32),
            pltpu.VMEM((qw, KNN), jnp.int32),
            pltpu.VMEM((qw * D + 16,), jnp.float32),
            pltpu.VMEM((qw * D * KNN,), jnp.float32),
        ],
    )
    def k(keys_hbm, kinds_hbm, q_hbm, out_hbm, keys_v, kinds_v, q_v, out_v):
        wid = lax.axis_index("s") * nc + lax.axis_index("c")
        base = wid * qw
        pltpu.sync_copy(keys_hbm, keys_v)
        pltpu.sync_copy(kinds_hbm.at[pl.ds(base, qw)], kinds_v)
        pltpu.sync_copy(q_hbm.at[pl.ds(base * D, qw * D)],
                        q_v.at[pl.ds(0, qw * D)])

        lane = lax.iota(jnp.int32, 16)

        def body(qi, c):
            qrow = q_v[pl.ds(qi * D, 16)]
            obase = lane + qi * (D * KNN)
            for g in range(KNN // 16):
                kv = kinds_v[qi, pl.ds(16 * g, 16)]
                kv3 = kv * 3
                for d in range(D):
                    val = plsc.load_gather(keys_v, [kv3 + d])
                    plsc.store_scatter(out_v, [obase + (d * KNN + 16 * g)],
                                       val - qrow[d])
            return c

        lax.fori_loop(0, qw, body, 0)
        pltpu.sync_copy(out_v, out_hbm.at[pl.ds(base * D * KNN, qw * D * KNN)])

    return k(keys_flat, k_inds, queries)


def _mlp_body(x_ref, w0_ref, b0_ref, w1_ref, b1_ref, w2_ref, b2_ref,
              wo_ref, bo_ref, o_ref):
    h = jnp.maximum(jnp.dot(x_ref[...], w0_ref[...],
                            preferred_element_type=jnp.float32) + b0_ref[...], 0.0)
    h = jnp.maximum(jnp.dot(h, w1_ref[...],
                            preferred_element_type=jnp.float32) + b1_ref[...], 0.0)
    h = jnp.maximum(jnp.dot(h, w2_ref[...],
                            preferred_element_type=jnp.float32) + b2_ref[...], 0.0)
    o_ref[...] = jnp.dot(h, wo_ref[...],
                         preferred_element_type=jnp.float32) + bo_ref[...]


def _mlp(x, w0, b0, w1, b1, w2, b2, wo, bo):
    full = lambda shape: pl.BlockSpec(shape, lambda i: tuple(0 for _ in shape))
    return pl.pallas_call(
        _mlp_body,
        grid=(Q // QB2,),
        in_specs=[
            pl.BlockSpec((QB2, KNN * D), lambda i: (i, 0)),
            full((KNN * D, HID)), full((HID,)),
            full((HID, HID)), full((HID,)),
            full((HID, HID)), full((HID,)),
            full((HID, N_OUT * D * EV)), full((N_OUT * D * EV,)),
        ],
        out_specs=pl.BlockSpec((QB2, N_OUT * D * EV), lambda i: (i, 0)),
        out_shape=jax.ShapeDtypeStruct((Q, N_OUT * D * EV), jnp.float32),
    )(x, w0, b0, w1, b1, w2, b2, wo, bo)


def kernel(queries, keys, W0, b0, W1, b1, W2, b2, Wout, bout):
    q2 = jnp.sum(queries * queries, axis=-1, keepdims=True)
    k2 = jnp.sum(keys * keys, axis=-1)
    k2b = jnp.broadcast_to(k2[None, :], (8, K))
    k_dists, k_inds = _topk(queries, keys.T, q2, k2b)
    flat_t = _sc_gather(keys.reshape(-1), k_inds,
                        queries.reshape(-1)).reshape(Q, D * KNN)
    W0_t = W0.reshape(KNN, D, HID).transpose(1, 0, 2).reshape(KNN * D, HID)
    out = _mlp(flat_t, W0_t, b0, W1, b1, W2, b2, Wout, bout)
    params = out.reshape(Q, N_OUT, D, EV)
    return params, k_inds, k_dists


# speedup vs baseline: 1.4435x; 1.1830x over previous
import functools

import jax
import jax.numpy as jnp
from jax import lax
from jax.experimental import pallas as pl
from jax.experimental.pallas import tpu as pltpu
from jax.experimental.pallas import tpu_sc as plsc

Q, K, D = 4096, 16384, 3
KNN = 32
HID = 1024
N_OUT, EV = 32, 2

QB = 256
QB2 = 512


CS = 512
NC_ = K // CS
TPC = 8
NCAND = NC_ * TPC


def _dist_into(q_ref, kt_ref, q2_ref, k2_ref, dist_ref):
    q = q_ref[...]
    kt = kt_ref[...]
    qk = jax.lax.dot_general(q, kt, (((1,), (0,)), ((), ())),
                             preferred_element_type=jnp.float32)
    q2 = q2_ref[...]
    k2 = k2_ref[0:1, :]
    dist_ref[...] = (q2 + k2) - 2.0 * qk


def _topk_body(q_ref, kt_ref, q2_ref, k2_ref, od_ref, oi_ref, dist_ref):
    _dist_into(q_ref, kt_ref, q2_ref, k2_ref, dist_ref)

    inf = jnp.float32(jnp.inf)
    iota = lax.broadcasted_iota(jnp.int32, (QB, K), 1)
    lane32 = lax.broadcasted_iota(jnp.int32, (QB, KNN), 1)

    def body(j, carry):
        od, oi = carry
        d = dist_ref[...]
        m = jnp.min(d, axis=1)
        idx = jnp.min(jnp.where(d == m[:, None], iota, K), axis=1)
        dist_ref[...] = jnp.where(iota == idx[:, None], inf, d)
        od = jnp.where(lane32 == j, m[:, None], od)
        oi = jnp.where(lane32 == j, idx[:, None], oi)
        return od, oi

    od0 = jnp.zeros((QB, KNN), jnp.float32)
    oi0 = jnp.zeros((QB, KNN), jnp.int32)
    od, oi = lax.fori_loop(0, KNN, body, (od0, oi0))
    od_ref[...] = od
    oi_ref[...] = oi


def _topk(queries, keys_t, q2, k2b):
    return pl.pallas_call(
        _topk_body,
        grid=(Q // QB,),
        in_specs=[
            pl.BlockSpec((QB, D), lambda i: (i, 0)),
            pl.BlockSpec((D, K), lambda i: (0, 0)),
            pl.BlockSpec((QB, 1), lambda i: (i, 0)),
            pl.BlockSpec((8, K), lambda i: (0, 0)),
        ],
        out_specs=[
            pl.BlockSpec((QB, KNN), lambda i: (i, 0)),
            pl.BlockSpec((QB, KNN), lambda i: (i, 0)),
        ],
        out_shape=[
            jax.ShapeDtypeStruct((Q, KNN), jnp.float32),
            jax.ShapeDtypeStruct((Q, KNN), jnp.int32),
        ],
        scratch_shapes=[pltpu.VMEM((QB, K), jnp.float32)],
    )(queries, keys_t, q2, k2b)


def _sc_gather(keys_flat, k_inds, queries):
    info = plsc.get_sparse_core_info()
    nc, ns = info.num_cores, info.num_subcores
    nw = nc * ns
    qw = Q // nw
    mesh = plsc.VectorSubcoreMesh(core_axis_name="c", subcore_axis_name="s")

    @functools.partial(
        pl.kernel, mesh=mesh,
        compiler_params=pltpu.CompilerParams(needs_layout_passes=False),
        out_type=jax.ShapeDtypeStruct((Q * D * KNN,), jnp.float32),
        scratch_types=[
            pltpu.VMEM((K * D,), jnp.float32),
            pltpu.VMEM((qw, KNN), jnp.int32),
            pltpu.VMEM((qw * D + 16,), jnp.float32),
            pltpu.VMEM((qw * D * KNN,), jnp.float32),
        ],
    )
    def k(keys_hbm, kinds_hbm, q_hbm, out_hbm, keys_v, kinds_v, q_v, out_v):
        wid = lax.axis_index("s") * nc + lax.axis_index("c")
        base = wid * qw
        pltpu.sync_copy(keys_hbm, keys_v)
        pltpu.sync_copy(kinds_hbm.at[pl.ds(base, qw)], kinds_v)
        pltpu.sync_copy(q_hbm.at[pl.ds(base * D, qw * D)],
                        q_v.at[pl.ds(0, qw * D)])

        lane = lax.iota(jnp.int32, 16)

        def body(qi, c):
            qrow = q_v[pl.ds(qi * D, 16)]
            obase = lane + qi * (D * KNN)
            for g in range(KNN // 16):
                kv = kinds_v[qi, pl.ds(16 * g, 16)]
                kv3 = kv * 3
                for d in range(D):
                    val = plsc.load_gather(keys_v, [kv3 + d])
                    plsc.store_scatter(out_v, [obase + (d * KNN + 16 * g)],
                                       val - qrow[d])
            return c

        lax.fori_loop(0, qw, body, 0)
        pltpu.sync_copy(out_v, out_hbm.at[pl.ds(base * D * KNN, qw * D * KNN)])

    return k(keys_flat, k_inds, queries)


def _mlp_body(x_ref, w0_ref, b0_ref, w1_ref, b1_ref, w2_ref, b2_ref,
              wo_ref, bo_ref, o_ref):
    h = jnp.maximum(jnp.dot(x_ref[...], w0_ref[...],
                            preferred_element_type=jnp.float32) + b0_ref[...], 0.0)
    h = jnp.maximum(jnp.dot(h, w1_ref[...],
                            preferred_element_type=jnp.float32) + b1_ref[...], 0.0)
    h = jnp.maximum(jnp.dot(h, w2_ref[...],
                            preferred_element_type=jnp.float32) + b2_ref[...], 0.0)
    o_ref[...] = jnp.dot(h, wo_ref[...],
                         preferred_element_type=jnp.float32) + bo_ref[...]


def _mlp(x, w0, b0, w1, b1, w2, b2, wo, bo):
    full = lambda shape: pl.BlockSpec(shape, lambda i: tuple(0 for _ in shape))
    return pl.pallas_call(
        _mlp_body,
        grid=(Q // QB2,),
        in_specs=[
            pl.BlockSpec((QB2, KNN * D), lambda i: (i, 0)),
            full((KNN * D, HID)), full((HID,)),
            full((HID, HID)), full((HID,)),
            full((HID, HID)), full((HID,)),
            full((HID, N_OUT * D * EV)), full((N_OUT * D * EV,)),
        ],
        out_specs=pl.BlockSpec((QB2, N_OUT * D * EV), lambda i: (i, 0)),
        out_shape=jax.ShapeDtypeStruct((Q, N_OUT * D * EV), jnp.float32),
    )(x, w0, b0, w1, b1, w2, b2, wo, bo)


def kernel(queries, keys, W0, b0, W1, b1, W2, b2, Wout, bout):
    q2 = jnp.sum(queries * queries, axis=-1, keepdims=True)
    k2 = jnp.sum(keys * keys, axis=-1)
    k2b = jnp.broadcast_to(k2[None, :], (8, K))
    k_dists, k_inds = _topk(queries, keys.T, q2, k2b)
    flat_t = _sc_gather(keys.reshape(-1), k_inds,
                        queries.reshape(-1)).reshape(Q, D * KNN)
    W0_t = W0.reshape(KNN, D, HID).transpose(1, 0, 2).reshape(KNN * D, HID)
    out = _mlp(flat_t, W0_t, b0, W1, b1, W2, b2, Wout, bout)
    params = out.reshape(Q, N_OUT, D, EV)
    return params, k_inds, k_dists


# speedup vs baseline: 1.6512x; 1.1439x over previous
import functools

import jax
import jax.numpy as jnp
from jax import lax
from jax.experimental import pallas as pl
from jax.experimental.pallas import tpu as pltpu
from jax.experimental.pallas import tpu_sc as plsc

Q, K, D = 4096, 16384, 3
KNN = 32
HID = 1024
N_OUT, EV = 32, 2

QB = 256
QB2 = 512


CS = 512
NC_ = K // CS
TPC = 8
NCAND = NC_ * TPC


def _dist_into(q_ref, kt_ref, q2_ref, k2_ref, dist_ref):
    q = q_ref[...]
    kt = kt_ref[...]
    qk = jax.lax.dot_general(q, kt, (((1,), (0,)), ((), ())),
                             preferred_element_type=jnp.float32)
    q2 = q2_ref[...]
    k2 = k2_ref[0:1, :]
    dist_ref[...] = (q2 + k2) - 2.0 * qk


def _topk_body(q_ref, kt_ref, q2_ref, k2_ref, od_ref, oi_ref, dist_ref):
    _dist_into(q_ref, kt_ref, q2_ref, k2_ref, dist_ref)

    inf = jnp.float32(jnp.inf)
    iotaf = lax.broadcasted_iota(jnp.int32, (QB, K), 1).astype(jnp.float32)
    lane32 = lax.broadcasted_iota(jnp.int32, (QB, KNN), 1)
    fK = jnp.float32(K)

    def body(j, carry):
        od, oi = carry
        d = dist_ref[...]
        m = jnp.min(d, axis=1)
        idxf = jnp.min(jnp.where(d == m[:, None], iotaf, fK), axis=1)
        dist_ref[...] = jnp.where(iotaf == idxf[:, None], inf, d)
        od = jnp.where(lane32 == j, m[:, None], od)
        oi = jnp.where(lane32 == j, idxf.astype(jnp.int32)[:, None], oi)
        return od, oi

    od0 = jnp.zeros((QB, KNN), jnp.float32)
    oi0 = jnp.zeros((QB, KNN), jnp.int32)
    od, oi = lax.fori_loop(0, KNN, body, (od0, oi0))
    od_ref[...] = od
    oi_ref[...] = oi


def _topk(queries, keys_t, q2, k2b):
    return pl.pallas_call(
        _topk_body,
        grid=(Q // QB,),
        in_specs=[
            pl.BlockSpec((QB, D), lambda i: (i, 0)),
            pl.BlockSpec((D, K), lambda i: (0, 0)),
            pl.BlockSpec((QB, 1), lambda i: (i, 0)),
            pl.BlockSpec((8, K), lambda i: (0, 0)),
        ],
        out_specs=[
            pl.BlockSpec((QB, KNN), lambda i: (i, 0)),
            pl.BlockSpec((QB, KNN), lambda i: (i, 0)),
        ],
        out_shape=[
            jax.ShapeDtypeStruct((Q, KNN), jnp.float32),
            jax.ShapeDtypeStruct((Q, KNN), jnp.int32),
        ],
        scratch_shapes=[pltpu.VMEM((QB, K), jnp.float32)],
    )(queries, keys_t, q2, k2b)


def _sc_gather(keys_flat, k_inds, queries):
    info = plsc.get_sparse_core_info()
    nc, ns = info.num_cores, info.num_subcores
    nw = nc * ns
    qw = Q // nw
    mesh = plsc.VectorSubcoreMesh(core_axis_name="c", subcore_axis_name="s")

    @functools.partial(
        pl.kernel, mesh=mesh,
        compiler_params=pltpu.CompilerParams(needs_layout_passes=False),
        out_type=jax.ShapeDtypeStruct((Q * D * KNN,), jnp.float32),
        scratch_types=[
            pltpu.VMEM((K * D,), jnp.float32),
            pltpu.VMEM((qw, KNN), jnp.int32),
            pltpu.VMEM((qw * D + 16,), jnp.float32),
            pltpu.VMEM((qw * D * KNN,), jnp.float32),
        ],
    )
    def k(keys_hbm, kinds_hbm, q_hbm, out_hbm, keys_v, kinds_v, q_v, out_v):
        wid = lax.axis_index("s") * nc + lax.axis_index("c")
        base = wid * qw
        pltpu.sync_copy(keys_hbm, keys_v)
        pltpu.sync_copy(kinds_hbm.at[pl.ds(base, qw)], kinds_v)
        pltpu.sync_copy(q_hbm.at[pl.ds(base * D, qw * D)],
                        q_v.at[pl.ds(0, qw * D)])

        lane = lax.iota(jnp.int32, 16)

        def body(qi, c):
            qrow = q_v[pl.ds(qi * D, 16)]
            obase = lane + qi * (D * KNN)
            for g in range(KNN // 16):
                kv = kinds_v[qi, pl.ds(16 * g, 16)]
                kv3 = kv * 3
                for d in range(D):
                    val = plsc.load_gather(keys_v, [kv3 + d])
                    plsc.store_scatter(out_v, [obase + (d * KNN + 16 * g)],
                                       val - qrow[d])
            return c

        lax.fori_loop(0, qw, body, 0)
        pltpu.sync_copy(out_v, out_hbm.at[pl.ds(base * D * KNN, qw * D * KNN)])

    return k(keys_flat, k_inds, queries)


def _mlp_body(x_ref, w0_ref, b0_ref, w1_ref, b1_ref, w2_ref, b2_ref,
              wo_ref, bo_ref, o_ref):
    h = jnp.maximum(jnp.dot(x_ref[...], w0_ref[...],
                            preferred_element_type=jnp.float32) + b0_ref[...], 0.0)
    h = jnp.maximum(jnp.dot(h, w1_ref[...],
                            preferred_element_type=jnp.float32) + b1_ref[...], 0.0)
    h = jnp.maximum(jnp.dot(h, w2_ref[...],
                            preferred_element_type=jnp.float32) + b2_ref[...], 0.0)
    o_ref[...] = jnp.dot(h, wo_ref[...],
                         preferred_element_type=jnp.float32) + bo_ref[...]


def _mlp(x, w0, b0, w1, b1, w2, b2, wo, bo):
    full = lambda shape: pl.BlockSpec(shape, lambda i: tuple(0 for _ in shape))
    return pl.pallas_call(
        _mlp_body,
        grid=(Q // QB2,),
        in_specs=[
            pl.BlockSpec((QB2, KNN * D), lambda i: (i, 0)),
            full((KNN * D, HID)), full((HID,)),
            full((HID, HID)), full((HID,)),
            full((HID, HID)), full((HID,)),
            full((HID, N_OUT * D * EV)), full((N_OUT * D * EV,)),
        ],
        out_specs=pl.BlockSpec((QB2, N_OUT * D * EV), lambda i: (i, 0)),
        out_shape=jax.ShapeDtypeStruct((Q, N_OUT * D * EV), jnp.float32),
    )(x, w0, b0, w1, b1, w2, b2, wo, bo)


def kernel(queries, keys, W0, b0, W1, b1, W2, b2, Wout, bout):
    q2 = jnp.sum(queries * queries, axis=-1, keepdims=True)
    k2 = jnp.sum(keys * keys, axis=-1)
    k2b = jnp.broadcast_to(k2[None, :], (8, K))
    k_dists, k_inds = _topk(queries, keys.T, q2, k2b)
    flat_t = _sc_gather(keys.reshape(-1), k_inds,
                        queries.reshape(-1)).reshape(Q, D * KNN)
    W0_t = W0.reshape(KNN, D, HID).transpose(1, 0, 2).reshape(KNN * D, HID)
    out = _mlp(flat_t, W0_t, b0, W1, b1, W2, b2, Wout, bout)
    params = out.reshape(Q, N_OUT, D, EV)
    return params, k_inds, k_dists


# speedup vs baseline: 3.2442x; 1.9647x over previous
import functools

import jax
import jax.numpy as jnp
from jax import lax
from jax.experimental import pallas as pl
from jax.experimental.pallas import tpu as pltpu
from jax.experimental.pallas import tpu_sc as plsc

Q, K, D = 4096, 16384, 3
KNN = 32
HID = 1024
N_OUT, EV = 32, 2

QB = 256
QB2 = 512


CS = 512
NC_ = K // CS
TPC = 8
NCAND = NC_ * TPC


def _dist_into(q_ref, kt_ref, q2_ref, k2_ref, dist_ref):
    q = q_ref[...]
    kt = kt_ref[...]
    qk = jax.lax.dot_general(q, kt, (((1,), (0,)), ((), ())),
                             preferred_element_type=jnp.float32)
    q2 = q2_ref[...]
    k2 = k2_ref[0:1, :]
    dist_ref[...] = (q2 + k2) - 2.0 * qk


def _topk_body(q_ref, kt_ref, q2_ref, k2_ref, od_ref, oi_ref,
               dist_ref, cd_ref, ci_ref):
    _dist_into(q_ref, kt_ref, q2_ref, k2_ref, dist_ref)

    inf = jnp.float32(jnp.inf)
    lane32 = lax.broadcasted_iota(jnp.int32, (QB, KNN), 1)
    iota_sf = lax.broadcasted_iota(jnp.int32, (QB, CS), 1).astype(jnp.float32)
    fCS = jnp.float32(CS)

    for t in range(TPC):
        ms, gis = [], []
        for c in range(NC_):
            dc = dist_ref[:, c * CS:(c + 1) * CS]
            mc = jnp.min(dc, axis=1)
            lif = jnp.min(jnp.where(dc == mc[:, None], iota_sf, fCS), axis=1)
            if t + 1 < TPC:
                dist_ref[:, c * CS:(c + 1) * CS] = \
                    jnp.where(iota_sf == lif[:, None], inf, dc)
            ms.append(mc[:, None])
            gis.append((lif + jnp.float32(c * CS))[:, None])
        cd_ref[:, t * NC_:(t + 1) * NC_] = jnp.concatenate(ms, axis=1)
        ci_ref[:, t * NC_:(t + 1) * NC_] = jnp.concatenate(gis, axis=1)

    chunk_iota = lax.broadcasted_iota(jnp.int32, (QB, NC_), 1)
    fK = jnp.float32(K)

    def mbody(j, carry):
        od, oi, counts = carry
        cd = cd_ref[...]
        cif = ci_ref[...]
        m = jnp.min(cd, axis=1)
        tie = cd == m[:, None]
        gi = jnp.min(jnp.where(tie, cif, fK), axis=1)
        cd_ref[...] = jnp.where(tie & (cif == gi[:, None]), inf, cd)
        gii = gi.astype(jnp.int32)
        od = jnp.where(lane32 == j, m[:, None], od)
        oi = jnp.where(lane32 == j, gii[:, None], oi)
        counts = counts + (chunk_iota ==
                           lax.shift_right_logical(gii, 9)[:, None]).astype(jnp.int32)
        return od, oi, counts

    od0 = jnp.zeros((QB, KNN), jnp.float32)
    oi0 = jnp.zeros((QB, KNN), jnp.int32)
    cnt0 = jnp.zeros((QB, NC_), jnp.int32)
    od, oi, counts = lax.fori_loop(0, KNN, mbody, (od0, oi0, cnt0))
    od_ref[...] = od
    oi_ref[...] = oi

    suspect = jnp.max(jnp.where(counts >= TPC, 1, 0))

    @pl.when(suspect == 1)
    def _fallback():
        _dist_into(q_ref, kt_ref, q2_ref, k2_ref, dist_ref)
        iotaf = lax.broadcasted_iota(jnp.int32, (QB, K), 1).astype(jnp.float32)

        def fb(j, carry):
            fod, foi = carry
            d = dist_ref[...]
            m = jnp.min(d, axis=1)
            idxf = jnp.min(jnp.where(d == m[:, None], iotaf, fK), axis=1)
            dist_ref[...] = jnp.where(iotaf == idxf[:, None], inf, d)
            fod = jnp.where(lane32 == j, m[:, None], fod)
            foi = jnp.where(lane32 == j, idxf.astype(jnp.int32)[:, None], foi)
            return fod, foi

        fod, foi = lax.fori_loop(0, KNN, fb, (od0, oi0))
        od_ref[...] = fod
        oi_ref[...] = foi


def _topk(queries, keys_t, q2, k2b):
    return pl.pallas_call(
        _topk_body,
        grid=(Q // QB,),
        in_specs=[
            pl.BlockSpec((QB, D), lambda i: (i, 0)),
            pl.BlockSpec((D, K), lambda i: (0, 0)),
            pl.BlockSpec((QB, 1), lambda i: (i, 0)),
            pl.BlockSpec((8, K), lambda i: (0, 0)),
        ],
        out_specs=[
            pl.BlockSpec((QB, KNN), lambda i: (i, 0)),
            pl.BlockSpec((QB, KNN), lambda i: (i, 0)),
        ],
        out_shape=[
            jax.ShapeDtypeStruct((Q, KNN), jnp.float32),
            jax.ShapeDtypeStruct((Q, KNN), jnp.int32),
        ],
        scratch_shapes=[
            pltpu.VMEM((QB, K), jnp.float32),
            pltpu.VMEM((QB, NCAND), jnp.float32),
            pltpu.VMEM((QB, NCAND), jnp.float32),
        ],
    )(queries, keys_t, q2, k2b)


def _sc_gather(keys_flat, k_inds, queries):
    info = plsc.get_sparse_core_info()
    nc, ns = info.num_cores, info.num_subcores
    nw = nc * ns
    qw = Q // nw
    mesh = plsc.VectorSubcoreMesh(core_axis_name="c", subcore_axis_name="s")

    @functools.partial(
        pl.kernel, mesh=mesh,
        compiler_params=pltpu.CompilerParams(needs_layout_passes=False),
        out_type=jax.ShapeDtypeStruct((Q * D * KNN,), jnp.float32),
        scratch_types=[
            pltpu.VMEM((K * D,), jnp.float32),
            pltpu.VMEM((qw, KNN), jnp.int32),
            pltpu.VMEM((qw * D + 16,), jnp.float32),
            pltpu.VMEM((qw * D * KNN,), jnp.float32),
        ],
    )
    def k(keys_hbm, kinds_hbm, q_hbm, out_hbm, keys_v, kinds_v, q_v, out_v):
        wid = lax.axis_index("s") * nc + lax.axis_index("c")
        base = wid * qw
        pltpu.sync_copy(keys_hbm, keys_v)
        pltpu.sync_copy(kinds_hbm.at[pl.ds(base, qw)], kinds_v)
        pltpu.sync_copy(q_hbm.at[pl.ds(base * D, qw * D)],
                        q_v.at[pl.ds(0, qw * D)])

        lane = lax.iota(jnp.int32, 16)

        def body(qi, c):
            qrow = q_v[pl.ds(qi * D, 16)]
            obase = lane + qi * (D * KNN)
            for g in range(KNN // 16):
                kv = kinds_v[qi, pl.ds(16 * g, 16)]
                kv3 = kv * 3
                for d in range(D):
                    val = plsc.load_gather(keys_v, [kv3 + d])
                    plsc.store_scatter(out_v, [obase + (d * KNN + 16 * g)],
                                       val - qrow[d])
            return c

        lax.fori_loop(0, qw, body, 0)
        pltpu.sync_copy(out_v, out_hbm.at[pl.ds(base * D * KNN, qw * D * KNN)])

    return k(keys_flat, k_inds, queries)


def _mlp_body(x_ref, w0_ref, b0_ref, w1_ref, b1_ref, w2_ref, b2_ref,
              wo_ref, bo_ref, o_ref):
    h = jnp.maximum(jnp.dot(x_ref[...], w0_ref[...],
                            preferred_element_type=jnp.float32) + b0_ref[...], 0.0)
    h = jnp.maximum(jnp.dot(h, w1_ref[...],
                            preferred_element_type=jnp.float32) + b1_ref[...], 0.0)
    h = jnp.maximum(jnp.dot(h, w2_ref[...],
                            preferred_element_type=jnp.float32) + b2_ref[...], 0.0)
    o_ref[...] = jnp.dot(h, wo_ref[...],
                         preferred_element_type=jnp.float32) + bo_ref[...]


def _mlp(x, w0, b0, w1, b1, w2, b2, wo, bo):
    full = lambda shape: pl.BlockSpec(shape, lambda i: tuple(0 for _ in shape))
    return pl.pallas_call(
        _mlp_body,
        grid=(Q // QB2,),
        in_specs=[
            pl.BlockSpec((QB2, KNN * D), lambda i: (i, 0)),
            full((KNN * D, HID)), full((HID,)),
            full((HID, HID)), full((HID,)),
            full((HID, HID)), full((HID,)),
            full((HID, N_OUT * D * EV)), full((N_OUT * D * EV,)),
        ],
        out_specs=pl.BlockSpec((QB2, N_OUT * D * EV), lambda i: (i, 0)),
        out_shape=jax.ShapeDtypeStruct((Q, N_OUT * D * EV), jnp.float32),
    )(x, w0, b0, w1, b1, w2, b2, wo, bo)


def kernel(queries, keys, W0, b0, W1, b1, W2, b2, Wout, bout):
    q2 = jnp.sum(queries * queries, axis=-1, keepdims=True)
    k2 = jnp.sum(keys * keys, axis=-1)
    k2b = jnp.broadcast_to(k2[None, :], (8, K))
    k_dists, k_inds = _topk(queries, keys.T, q2, k2b)
    flat_t = _sc_gather(keys.reshape(-1), k_inds,
                        queries.reshape(-1)).reshape(Q, D * KNN)
    W0_t = W0.reshape(KNN, D, HID).transpose(1, 0, 2).reshape(KNN * D, HID)
    out = _mlp(flat_t, W0_t, b0, W1, b1, W2, b2, Wout, bout)
    params = out.reshape(Q, N_OUT, D, EV)
    return params, k_inds, k_dists


# speedup vs baseline: 3.3217x; 1.0239x over previous
import functools

import jax
import jax.numpy as jnp
from jax import lax
from jax.experimental import pallas as pl
from jax.experimental.pallas import tpu as pltpu
from jax.experimental.pallas import tpu_sc as plsc

Q, K, D = 4096, 16384, 3
KNN = 32
HID = 1024
N_OUT, EV = 32, 2

QB = 256
QB2 = 512


CS = 512
NC_ = K // CS
TPC = 8
NCAND = NC_ * TPC


def _dist_into(q_ref, kt_ref, q2_ref, k2_ref, dist_ref):
    q = q_ref[...]
    kt = kt_ref[...]
    qk = jax.lax.dot_general(q, kt, (((1,), (0,)), ((), ())),
                             preferred_element_type=jnp.float32)
    q2 = q2_ref[...]
    k2 = k2_ref[0:1, :]
    dist_ref[...] = (q2 + k2) - 2.0 * qk


def _phasea_body(q_ref, kt_ref, q2_ref, k2_ref, cd_ref, ci_ref, dist_ref):
    _dist_into(q_ref, kt_ref, q2_ref, k2_ref, dist_ref)

    inf = jnp.float32(jnp.inf)
    iota_sf = lax.broadcasted_iota(jnp.int32, (QB, CS), 1).astype(jnp.float32)
    fCS = jnp.float32(CS)

    for t in range(TPC):
        ms, gis = [], []
        for c in range(NC_):
            dc = dist_ref[:, c * CS:(c + 1) * CS]
            mc = jnp.min(dc, axis=1)
            lif = jnp.min(jnp.where(dc == mc[:, None], iota_sf, fCS), axis=1)
            if t + 1 < TPC:
                dist_ref[:, c * CS:(c + 1) * CS] = \
                    jnp.where(iota_sf == lif[:, None], inf, dc)
            ms.append(mc[:, None])
            gis.append((lif + jnp.float32(c * CS))[:, None])
        cd_ref[:, t * NC_:(t + 1) * NC_] = jnp.concatenate(ms, axis=1)
        ci_ref[:, t * NC_:(t + 1) * NC_] = jnp.concatenate(gis, axis=1)


def _merge_body(cd_ref, ci_ref, od_ref, oi_ref, sus_ref):
    inf = jnp.float32(jnp.inf)
    fK = jnp.float32(K)
    lane32 = lax.broadcasted_iota(jnp.int32, (Q, KNN), 1)
    chunk_iota = lax.broadcasted_iota(jnp.int32, (Q, NC_), 1)

    def mbody(j, carry):
        od, oi, counts = carry
        cd = cd_ref[...]
        cif = ci_ref[...]
        m = jnp.min(cd, axis=1)
        tie = cd == m[:, None]
        gi = jnp.min(jnp.where(tie, cif, fK), axis=1)
        cd_ref[...] = jnp.where(tie & (cif == gi[:, None]), inf, cd)
        gii = gi.astype(jnp.int32)
        od = jnp.where(lane32 == j, m[:, None], od)
        oi = jnp.where(lane32 == j, gii[:, None], oi)
        counts = counts + (chunk_iota ==
                           lax.shift_right_logical(gii, 9)[:, None]).astype(jnp.int32)
        return od, oi, counts

    od0 = jnp.zeros((Q, KNN), jnp.float32)
    oi0 = jnp.zeros((Q, KNN), jnp.int32)
    cnt0 = jnp.zeros((Q, NC_), jnp.int32)
    od, oi, counts = lax.fori_loop(0, KNN, mbody, (od0, oi0, cnt0))
    od_ref[...] = od
    oi_ref[...] = oi
    sus_ref[...] = jnp.max(jnp.where(counts >= TPC, 1, 0),
                           axis=1, keepdims=True)


def _fixup_body(q_ref, kt_ref, q2_ref, k2_ref, sus_ref, odi_ref, oii_ref,
                od_ref, oi_ref, dist_ref):
    od_ref[...] = odi_ref[...]
    oi_ref[...] = oii_ref[...]
    suspect = jnp.max(sus_ref[...])

    @pl.when(suspect >= 1)
    def _fallback():
        _dist_into(q_ref, kt_ref, q2_ref, k2_ref, dist_ref)
        inf = jnp.float32(jnp.inf)
        fK = jnp.float32(K)
        lane32 = lax.broadcasted_iota(jnp.int32, (QB, KNN), 1)
        iotaf = lax.broadcasted_iota(jnp.int32, (QB, K), 1).astype(jnp.float32)

        def fb(j, carry):
            fod, foi = carry
            d = dist_ref[...]
            m = jnp.min(d, axis=1)
            idxf = jnp.min(jnp.where(d == m[:, None], iotaf, fK), axis=1)
            dist_ref[...] = jnp.where(iotaf == idxf[:, None], inf, d)
            fod = jnp.where(lane32 == j, m[:, None], fod)
            foi = jnp.where(lane32 == j, idxf.astype(jnp.int32)[:, None], foi)
            return fod, foi

        fod, foi = lax.fori_loop(0, KNN, fb,
                                 (jnp.zeros((QB, KNN), jnp.float32),
                                  jnp.zeros((QB, KNN), jnp.int32)))
        od_ref[...] = fod
        oi_ref[...] = foi


def _topk(queries, keys_t, q2, k2b):
    cd, ci = pl.pallas_call(
        _phasea_body,
        grid=(Q // QB,),
        in_specs=[
            pl.BlockSpec((QB, D), lambda i: (i, 0)),
            pl.BlockSpec((D, K), lambda i: (0, 0)),
            pl.BlockSpec((QB, 1), lambda i: (i, 0)),
            pl.BlockSpec((8, K), lambda i: (0, 0)),
        ],
        out_specs=[
            pl.BlockSpec((QB, NCAND), lambda i: (i, 0)),
            pl.BlockSpec((QB, NCAND), lambda i: (i, 0)),
        ],
        out_shape=[
            jax.ShapeDtypeStruct((Q, NCAND), jnp.float32),
            jax.ShapeDtypeStruct((Q, NCAND), jnp.float32),
        ],
        scratch_shapes=[pltpu.VMEM((QB, K), jnp.float32)],
    )(queries, keys_t, q2, k2b)

    od, oi, sus = pl.pallas_call(
        _merge_body,
        out_shape=[
            jax.ShapeDtypeStruct((Q, KNN), jnp.float32),
            jax.ShapeDtypeStruct((Q, KNN), jnp.int32),
            jax.ShapeDtypeStruct((Q, 1), jnp.int32),
        ],
    )(cd, ci)

    od, oi = pl.pallas_call(
        _fixup_body,
        grid=(Q // QB,),
        in_specs=[
            pl.BlockSpec((QB, D), lambda i: (i, 0)),
            pl.BlockSpec((D, K), lambda i: (0, 0)),
            pl.BlockSpec((QB, 1), lambda i: (i, 0)),
            pl.BlockSpec((8, K), lambda i: (0, 0)),
            pl.BlockSpec((QB, 1), lambda i: (i, 0)),
            pl.BlockSpec((QB, KNN), lambda i: (i, 0)),
            pl.BlockSpec((QB, KNN), lambda i: (i, 0)),
        ],
        out_specs=[
            pl.BlockSpec((QB, KNN), lambda i: (i, 0)),
            pl.BlockSpec((QB, KNN), lambda i: (i, 0)),
        ],
        out_shape=[
            jax.ShapeDtypeStruct((Q, KNN), jnp.float32),
            jax.ShapeDtypeStruct((Q, KNN), jnp.int32),
        ],
        scratch_shapes=[pltpu.VMEM((QB, K), jnp.float32)],
    )(queries, keys_t, q2, k2b, sus, od, oi)
    return od, oi


def _sc_gather(keys_flat, k_inds, queries):
    info = plsc.get_sparse_core_info()
    nc, ns = info.num_cores, info.num_subcores
    nw = nc * ns
    qw = Q // nw
    mesh = plsc.VectorSubcoreMesh(core_axis_name="c", subcore_axis_name="s")

    @functools.partial(
        pl.kernel, mesh=mesh,
        compiler_params=pltpu.CompilerParams(needs_layout_passes=False),
        out_type=jax.ShapeDtypeStruct((Q * D * KNN,), jnp.float32),
        scratch_types=[
            pltpu.VMEM((K * D,), jnp.float32),
            pltpu.VMEM((qw, KNN), jnp.int32),
            pltpu.VMEM((qw * D + 16,), jnp.float32),
            pltpu.VMEM((qw * D * KNN,), jnp.float32),
        ],
    )
    def k(keys_hbm, kinds_hbm, q_hbm, out_hbm, keys_v, kinds_v, q_v, out_v):
        wid = lax.axis_index("s") * nc + lax.axis_index("c")
        base = wid * qw
        pltpu.sync_copy(keys_hbm, keys_v)
        pltpu.sync_copy(kinds_hbm.at[pl.ds(base, qw)], kinds_v)
        pltpu.sync_copy(q_hbm.at[pl.ds(base * D, qw * D)],
                        q_v.at[pl.ds(0, qw * D)])

        lane = lax.iota(jnp.int32, 16)

        def body(qi, c):
            qrow = q_v[pl.ds(qi * D, 16)]
            obase = lane + qi * (D * KNN)
            for g in range(KNN // 16):
                kv = kinds_v[qi, pl.ds(16 * g, 16)]
                kv3 = kv * 3
                for d in range(D):
                    val = plsc.load_gather(keys_v, [kv3 + d])
                    plsc.store_scatter(out_v, [obase + (d * KNN + 16 * g)],
                                       val - qrow[d])
            return c

        lax.fori_loop(0, qw, body, 0)
        pltpu.sync_copy(out_v, out_hbm.at[pl.ds(base * D * KNN, qw * D * KNN)])

    return k(keys_flat, k_inds, queries)


def _mlp_body(x_ref, w0_ref, b0_ref, w1_ref, b1_ref, w2_ref, b2_ref,
              wo_ref, bo_ref, o_ref):
    h = jnp.maximum(jnp.dot(x_ref[...], w0_ref[...],
                            preferred_element_type=jnp.float32) + b0_ref[...], 0.0)
    h = jnp.maximum(jnp.dot(h, w1_ref[...],
                            preferred_element_type=jnp.float32) + b1_ref[...], 0.0)
    h = jnp.maximum(jnp.dot(h, w2_ref[...],
                            preferred_element_type=jnp.float32) + b2_ref[...], 0.0)
    o_ref[...] = jnp.dot(h, wo_ref[...],
                         preferred_element_type=jnp.float32) + bo_ref[...]


def _mlp(x, w0, b0, w1, b1, w2, b2, wo, bo):
    full = lambda shape: pl.BlockSpec(shape, lambda i: tuple(0 for _ in shape))
    return pl.pallas_call(
        _mlp_body,
        grid=(Q // QB2,),
        in_specs=[
            pl.BlockSpec((QB2, KNN * D), lambda i: (i, 0)),
            full((KNN * D, HID)), full((HID,)),
            full((HID, HID)), full((HID,)),
            full((HID, HID)), full((HID,)),
            full((HID, N_OUT * D * EV)), full((N_OUT * D * EV,)),
        ],
        out_specs=pl.BlockSpec((QB2, N_OUT * D * EV), lambda i: (i, 0)),
        out_shape=jax.ShapeDtypeStruct((Q, N_OUT * D * EV), jnp.float32),
    )(x, w0, b0, w1, b1, w2, b2, wo, bo)


def kernel(queries, keys, W0, b0, W1, b1, W2, b2, Wout, bout):
    q2 = jnp.sum(queries * queries, axis=-1, keepdims=True)
    k2 = jnp.sum(keys * keys, axis=-1)
    k2b = jnp.broadcast_to(k2[None, :], (8, K))
    k_dists, k_inds = _topk(queries, keys.T, q2, k2b)
    flat_t = _sc_gather(keys.reshape(-1), k_inds,
                        queries.reshape(-1)).reshape(Q, D * KNN)
    W0_t = W0.reshape(KNN, D, HID).transpose(1, 0, 2).reshape(KNN * D, HID)
    out = _mlp(flat_t, W0_t, b0, W1, b1, W2, b2, Wout, bout)
    params = out.reshape(Q, N_OUT, D, EV)
    return params, k_inds, k_dists
